# Initial kernel scaffold; baseline (speedup 1.0000x reference)
import jax
import jax.numpy as jnp
from jax.experimental import pallas as pl


def kernel(heatmap, points):
    raise NotImplementedError("write your pallas kernel here")


# speedup vs baseline: 1.3598x; 1.3598x over previous
import functools

import jax
import jax.numpy as jnp
from jax import lax
from jax.experimental import pallas as pl
from jax.experimental.pallas import tpu as pltpu
from jax.experimental.pallas import tpu_sc as plsc

_NUM_CORES = 2
_NUM_SUBCORES = 16
_LANES = 16
_NW = _NUM_CORES * _NUM_SUBCORES

_CHUNK = 32768
_UNROLL = 4

import numpy as np

_NEG = np.float32(-np.inf)
_POS = np.float32(np.inf)
_BIGI = np.int32(1 << 30)


def _insert_topk(x, ix, t1, t2, t3, i1, i2, i3):
    c1 = x > t1
    nt1 = jnp.maximum(x, t1)
    l1 = jnp.minimum(x, t1)
    ni1 = jnp.where(c1, ix, i1)
    si1 = jnp.where(c1, i1, ix)
    c2 = l1 > t2
    nt2 = jnp.maximum(l1, t2)
    l2 = jnp.minimum(l1, t2)
    ni2 = jnp.where(c2, si1, i2)
    si2 = jnp.where(c2, i2, si1)
    c3 = l2 > t3
    nt3 = jnp.maximum(l2, t3)
    ni3 = jnp.where(c3, si2, i3)
    return nt1, nt2, nt3, ni1, ni2, ni3


@functools.lru_cache(maxsize=None)
def _build(B, L, N):
    rows_per_worker = (B * L) // _NW
    nchunk = N // _CHUNK
    steps = _CHUNK // _LANES // _UNROLL
    total = rows_per_worker * nchunk

    mesh = plsc.VectorSubcoreMesh(core_axis_name="c", subcore_axis_name="s")

    @functools.partial(
        pl.kernel,
        mesh=mesh,
        out_type=jax.ShapeDtypeStruct((B, 2 * _LANES), jnp.float32),
        scratch_types=[
            pltpu.VMEM((_CHUNK,), jnp.float32),
            pltpu.VMEM((_CHUNK,), jnp.float32),
            pltpu.VMEM((_LANES,), jnp.int32),
            pltpu.VMEM((_LANES,), jnp.float32),
            pltpu.VMEM((1, 2 * _LANES), jnp.float32),
            pltpu.SemaphoreType.DMA,
            pltpu.SemaphoreType.DMA,
            pltpu.SemaphoreType.DMA,
        ],
        compiler_params=pltpu.CompilerParams(needs_layout_passes=False),
    )
    def sc_kernel(hm_hbm, pts_hbm, out_hbm, buf0, buf1, gidx, gbuf, res,
                  sem0, sem1, semg):
        wid = lax.axis_index("s") * _NUM_CORES + lax.axis_index("c")
        bufs = (buf0, buf1)
        sems = (sem0, sem1)
        iota = lax.iota(jnp.int32, _LANES)

        def start_copy(k):
            r, c = divmod(k, nchunk)
            base = (wid * L + r) * N + c * _CHUNK
            cp = pltpu.make_async_copy(
                hm_hbm.at[pl.ds(base, _CHUNK)], bufs[k % 2], sems[k % 2])
            cp.start()
            return cp

        def run_chunk(buf, c, acc):
            iotas = [iota + (c * _CHUNK + j * _LANES) for j in range(_UNROLL)]

            def step(s, acc):
                t1, t2, t3, i1, i2, i3, mn = acc
                s_off = s * (_UNROLL * _LANES)
                for j in range(_UNROLL):
                    x = buf[pl.ds(s_off + j * _LANES, _LANES)]
                    ix = iotas[j] + s_off
                    mn = jnp.minimum(mn, x)
                    t1, t2, t3, i1, i2, i3 = _insert_topk(
                        x, ix, t1, t2, t3, i1, i2, i3)
                return (t1, t2, t3, i1, i2, i3, mn)

            return lax.fori_loop(0, steps, step, acc)

        def finalize_row(acc):
            t1, t2, t3, i1, i2, i3, mn = acc
            mn_s = jnp.min(mn)
            vals = []
            idcs = []
            for _ in range(3):
                m = jnp.maximum(jnp.maximum(jnp.max(t1), jnp.max(t2)),
                                jnp.max(t3))
                c1 = t1 == m
                c2 = t2 == m
                c3 = t3 == m
                mi = jnp.minimum(
                    jnp.minimum(jnp.min(jnp.where(c1, i1, _BIGI)),
                                jnp.min(jnp.where(c2, i2, _BIGI))),
                    jnp.min(jnp.where(c3, i3, _BIGI)))
                vals.append(m)
                idcs.append(mi)
                t1 = jnp.where(c1 & (i1 == mi), _NEG, t1)
                t2 = jnp.where(c2 & (i2 == mi), _NEG, t2)
                t3 = jnp.where(c3 & (i3 == mi), _NEG, t3)
            v1, v2, v3 = vals
            rng = v1 - mn_s
            tsum = (v1 - mn_s) + (v2 - mn_s) + (v3 - mn_s)
            denom = tsum + jnp.float32(1e-9) * rng

            pk = iota // 3
            dd = iota - pk * 3
            sel = jnp.where(pk == 0, idcs[0],
                            jnp.where(pk == 1, idcs[1], idcs[2]))
            gidx[...] = (wid * N + sel) * 3 + dd
            pltpu.make_async_copy(pts_hbm.at[gidx], gbuf, semg).start()
            pltpu.make_async_copy(pts_hbm.at[gidx], gbuf, semg).wait()
            avec = jnp.where(pk == 0, v1, jnp.where(pk == 1, v2, v3)) - mn_s
            dvec = (avec * gbuf[...]) / denom
            return [dvec[d] + dvec[3 + d] + dvec[6 + d] for d in range(3)]

        cp = start_copy(0)
        acc = None
        lms = []
        for k in range(total):
            nxt = start_copy(k + 1) if k + 1 < total else None
            cp.wait()
            r, c = divmod(k, nchunk)
            if c == 0:
                neg = jnp.full((_LANES,), _NEG, jnp.float32)
                zero = jnp.zeros((_LANES,), jnp.int32)
                acc = (neg, neg, neg, zero, zero, zero,
                       jnp.full((_LANES,), _POS, jnp.float32))
            acc = run_chunk(bufs[k % 2], c, acc)
            if c == nchunk - 1:
                lms.extend(finalize_row(acc))
            cp = nxt
        for half in range(2):
            v = jnp.zeros((_LANES,), jnp.float32)
            for p in range(_LANES):
                idx = half * _LANES + p
                if idx < len(lms):
                    v = jnp.where(iota == p, lms[idx], v)
            res[0, pl.ds(half * _LANES, _LANES)] = v
        pltpu.sync_copy(res, out_hbm.at[pl.ds(wid, 1)])

    return sc_kernel


def kernel(heatmap, points):
    B, L, N = heatmap.shape
    hm_flat = heatmap.reshape(B * L * N)
    pts_flat = points.reshape(B * N * 3)
    padded = _build(B, L, N)(hm_flat, pts_flat)
    return padded[:, : L * 3].reshape(B, L, 3)
